# Initial kernel scaffold; baseline (speedup 1.0000x reference)
import jax
import jax.numpy as jnp
from jax.experimental import pallas as pl


def kernel(atom_weights, atom_in_fea, bond_nbr_fea, self_fea_idx, nbr_fea_idx, W_filter, b_filter, gamma_filter, beta_filter, W_core, b_core, gamma_core, beta_core, W_gate, b_gate):
    raise NotImplementedError("write your pallas kernel here")


# speedup vs baseline: 1.8420x; 1.8420x over previous
import dataclasses
import functools

import jax
import jax.numpy as jnp
from jax import lax
from jax.experimental import pallas as pl
from jax.experimental.pallas import tpu as pltpu
from jax.experimental.pallas import tpu_sc as plsc

N = 10000
M = 320000
AF = 128
NF = 16

NC, NS = 2, 16
NWORK = NC * NS
GW = 80

HALF = 5120
ACCROWS = 5136
EB = 512
NEB = M // EB

_mesh = plsc.VectorSubcoreMesh(core_axis_name="c", subcore_axis_name="s")

_sc_params = pltpu.CompilerParams()
if "needs_layout_passes" in pltpu.CompilerParams.__dataclass_fields__:
    _sc_params = dataclasses.replace(_sc_params, needs_layout_passes=False)


def _k0_body(a_ref, wnode_ref, ps_ref, pn_ref):
    a = a_ref[...]
    ps_ref[...] = jnp.dot(a, wnode_ref[...][:, :256],
                          preferred_element_type=jnp.float32)
    pn_ref[...] = jnp.dot(a, wnode_ref[...][:, 256:],
                          preferred_element_type=jnp.float32)


def _make_tables(atom_in_fea, w_node):
    R = 400
    return pl.pallas_call(
        _k0_body,
        grid=(N // R,),
        in_specs=[
            pl.BlockSpec((R, AF), lambda i: (i, 0)),
            pl.BlockSpec((AF, 512), lambda i: (0, 0)),
        ],
        out_specs=[
            pl.BlockSpec((R, 256), lambda i: (i, 0)),
            pl.BlockSpec((R, 256), lambda i: (i, 0)),
        ],
        out_shape=[
            jax.ShapeDtypeStruct((N, 256), jnp.float32),
            jax.ShapeDtypeStruct((N, 256), jnp.float32),
        ],
    )(atom_in_fea, w_node)


@functools.partial(
    pl.kernel,
    mesh=_mesh,
    compiler_params=_sc_params,
    out_type=[
        jax.ShapeDtypeStruct((M, 256), jnp.float32),
        jax.ShapeDtypeStruct((M,), jnp.float32),
    ],
    scratch_types=[
        pltpu.VMEM((GW,), jnp.int32),
        pltpu.VMEM((GW,), jnp.int32),
        pltpu.VMEM((GW, 256), jnp.float32),
        pltpu.VMEM((GW, 256), jnp.float32),
        pltpu.VMEM((GW,), jnp.float32),
        pltpu.VMEM((N,), jnp.float32),
        pltpu.SemaphoreType.DMA,
        pltpu.SemaphoreType.DMA,
    ],
)
def _k1_gather(ps_hbm, pn_hbm, si_hbm, ni_hbm, aw_hbm, xpre_hbm, wg_hbm,
               si_v, ni_v, a_v, b_v, w_v, wtab_v, sem_a, sem_b):
    wid = lax.axis_index("s") * NC + lax.axis_index("c")
    per_w = M // NWORK
    nwin = per_w // GW
    base = wid * per_w

    pltpu.sync_copy(aw_hbm, wtab_v)

    @pl.loop(0, nwin)
    def _win(w):
        off = base + w * GW
        pltpu.sync_copy(si_hbm.at[pl.ds(off, GW)], si_v)
        pltpu.sync_copy(ni_hbm.at[pl.ds(off, GW)], ni_v)
        cp_a = pltpu.async_copy(ps_hbm.at[si_v], a_v, sem_a)
        cp_b = pltpu.async_copy(pn_hbm.at[ni_v], b_v, sem_b)
        cp_a.wait()
        cp_b.wait()

        @pl.loop(0, GW)
        def _row(r):
            @pl.loop(0, 256, step=16)
            def _chunk(cc):
                b_v[r, pl.ds(cc, 16)] = (b_v[r, pl.ds(cc, 16)]
                                         + a_v[r, pl.ds(cc, 16)])

        @pl.loop(0, GW, step=16)
        def _wchunk(cc):
            idx = ni_v[pl.ds(cc, 16)]
            w_v[pl.ds(cc, 16)] = plsc.load_gather(wtab_v, [idx])

        pltpu.sync_copy(b_v, xpre_hbm.at[pl.ds(off, GW)])
        pltpu.sync_copy(w_v, wg_hbm.at[pl.ds(off, GW)])


def _k2_body(xpre_ref, bond_ref, wb_ref, acc_ref):
    @pl.when(pl.program_id(0) == 0)
    def _init():
        acc_ref[...] = jnp.zeros_like(acc_ref)

    x = xpre_ref[...] + jnp.dot(
        bond_ref[...], wb_ref[...], preferred_element_type=jnp.float32)
    acc_ref[0:1, :] += jnp.sum(x, axis=0, keepdims=True)
    acc_ref[1:2, :] += jnp.sum(x * x, axis=0, keepdims=True)


def _bn_stats(xpre, bond, wb_cat):
    return pl.pallas_call(
        _k2_body,
        grid=(NEB,),
        in_specs=[
            pl.BlockSpec((EB, 256), lambda i: (i, 0)),
            pl.BlockSpec((EB, NF), lambda i: (i, 0)),
            pl.BlockSpec((NF, 256), lambda i: (0, 0)),
        ],
        out_specs=pl.BlockSpec((8, 256), lambda i: (0, 0)),
        out_shape=jax.ShapeDtypeStruct((8, 256), jnp.float32),
    )(xpre, bond, wb_cat)


def _k3_body(xpre_ref, bond_ref, wcol_ref, wb_ref, stats_ref, gb_ref, wg_ref,
             y_ref):
    inv_m = 1.0 / M
    mu = stats_ref[0:1, :] * inv_m
    ex2 = stats_ref[1:2, :] * inv_m
    var = ex2 - mu * mu
    rstd = lax.rsqrt(var + 1e-5)
    a_aff = rstd * gb_ref[0:1, :]
    c_aff = gb_ref[1:2, :] - mu * a_aff

    x = xpre_ref[...] + jnp.dot(
        bond_ref[...], wb_ref[...], preferred_element_type=jnp.float32)
    xn = x * a_aff + c_aff
    xf = xn[:, :AF]
    xc = xn[:, AF:]
    f = jax.nn.sigmoid(xf)
    e = jnp.where(xc > 0, xc, jnp.exp(jnp.minimum(xc, 0.0)) - 1.0)
    msg = f * e

    g = jnp.dot(msg, wg_ref[...], preferred_element_type=jnp.float32)
    g = g[:, 0:1] + gb_ref[2:3, 0:1]
    t = wcol_ref[...] * jnp.exp(jnp.minimum(g, 50.0))
    pad = jnp.zeros((msg.shape[0], 127), jnp.float32)
    y_ref[...] = jnp.concatenate([t * msg, t, pad], axis=1)


def _messages(xpre, bond, wcol, wb_cat, stats, gb, wg):
    return pl.pallas_call(
        _k3_body,
        grid=(NEB,),
        in_specs=[
            pl.BlockSpec((EB, 256), lambda i: (i, 0)),
            pl.BlockSpec((EB, NF), lambda i: (i, 0)),
            pl.BlockSpec((EB, 1), lambda i: (i, 0)),
            pl.BlockSpec((NF, 256), lambda i: (0, 0)),
            pl.BlockSpec((8, 256), lambda i: (0, 0)),
            pl.BlockSpec((8, 256), lambda i: (0, 0)),
            pl.BlockSpec((AF, 1), lambda i: (0, 0)),
        ],
        out_specs=pl.BlockSpec((EB, 256), lambda i: (i, 0)),
        out_shape=jax.ShapeDtypeStruct((M, 256), jnp.float32),
    )(xpre, bond, wcol, wb_cat, stats, gb, wg)


NPAD = 2 * HALF
NODES_PER_W = NPAD // NWORK
ACCTOT = NPAD


@functools.partial(
    pl.kernel,
    mesh=_mesh,
    compiler_params=_sc_params,
    out_type=jax.ShapeDtypeStruct((NPAD, 256), jnp.float32),
    scratch_types=[
        pltpu.VMEM((GW, 256), jnp.float32),
        pltpu.VMEM((GW,), jnp.int32),
        pltpu.VMEM((NODES_PER_W + 8, 256), jnp.float32),
    ],
)
def _k4_scatter(y_hbm, si_hbm, out_hbm, y_v, si_v, acc_v):
    cid = lax.axis_index("c")
    sid = lax.axis_index("s")
    wid = cid * NS + sid
    nwin = M // GW
    nlo = wid * NODES_PER_W
    nhi = nlo + NODES_PER_W
    ii16 = lax.iota(jnp.int32, 16)

    @pl.loop(0, NODES_PER_W + 8)
    def _zr(r):
        @pl.loop(0, 256, step=16)
        def _zc(cc):
            acc_v[r, pl.ds(cc, 16)] = jnp.zeros((16,), jnp.float32)

    @pl.loop(0, nwin)
    def _win(w):
        off = w * GW
        pltpu.sync_copy(si_hbm.at[pl.ds(off, GW)], si_v)
        first = jnp.min(si_v[pl.ds(0, 16)])
        last = jnp.max(si_v[pl.ds(GW - 16, 16)])

        @pl.when(jnp.logical_and(last >= nlo, first < nhi))
        def _accum():
            pltpu.sync_copy(y_hbm.at[pl.ds(off, GW)], y_v)

            @pl.loop(0, GW, step=16)
            def _ec(ec):
                sic = si_v[pl.ds(ec, 16)]
                loc = sic - nlo
                inr = jnp.logical_and(loc >= 0, loc < NODES_PER_W)
                rowc = jnp.where(inr, loc, NODES_PER_W)
                for e in range(16):
                    row_e = jnp.sum(jnp.where(ii16 == e, rowc, 0))
                    rows = jnp.broadcast_to(row_e, (16,))

                    @pl.loop(0, 256, step=16)
                    def _ck(k):
                        v = y_v[ec + e, pl.ds(k, 16)]
                        plsc.addupdate_scatter(acc_v, [rows, k + ii16], v)

    @pl.loop(0, NODES_PER_W // GW)
    def _dump(k):
        pltpu.sync_copy(acc_v.at[pl.ds(k * GW, GW)],
                        out_hbm.at[pl.ds(nlo + k * GW, GW)])


def _k5_body(parts_ref, out_ref):
    s = parts_ref[...]
    out_ref[...] = s[:, :AF] / (s[:, AF:AF + 1] + 1e-13)


def _finalize(parts):
    R = 80
    return pl.pallas_call(
        _k5_body,
        grid=(N // R,),
        in_specs=[pl.BlockSpec((R, 256), lambda i: (i, 0))],
        out_specs=pl.BlockSpec((R, AF), lambda i: (i, 0)),
        out_shape=jax.ShapeDtypeStruct((N, AF), jnp.float32),
    )(parts)


def kernel(atom_weights, atom_in_fea, bond_nbr_fea, self_fea_idx, nbr_fea_idx,
           W_filter, b_filter, gamma_filter, beta_filter,
           W_core, b_core, gamma_core, beta_core,
           W_gate, b_gate):
    f32 = jnp.float32
    w_node = jnp.concatenate(
        [W_filter[:AF], W_core[:AF], W_filter[AF:2 * AF], W_core[AF:2 * AF]],
        axis=1).astype(f32)
    wb_cat = jnp.concatenate(
        [W_filter[2 * AF:], W_core[2 * AF:]], axis=1).astype(f32)
    gamma_cat = jnp.concatenate([gamma_filter, gamma_core])[None, :]
    beta_cat = jnp.concatenate([beta_filter, beta_core])[None, :]
    bg_row = jnp.broadcast_to(b_gate.reshape(1, 1), (1, 256))
    gb = jnp.concatenate(
        [gamma_cat, beta_cat, bg_row,
         jnp.zeros((5, 256), f32)], axis=0).astype(f32)

    ps, pn = _make_tables(atom_in_fea.astype(f32), w_node)
    xpre, wg = _k1_gather(ps, pn, self_fea_idx, nbr_fea_idx,
                          atom_weights.reshape(N).astype(f32))
    bond = bond_nbr_fea.astype(f32)
    stats = _bn_stats(xpre, bond, wb_cat)
    y = _messages(xpre, bond, wg.reshape(M, 1), wb_cat, stats, gb,
                  W_gate.astype(f32))
    parts = _k4_scatter(y, self_fea_idx)
    return _finalize(parts)


# speedup vs baseline: 2.8860x; 1.5667x over previous
import dataclasses
import functools

import jax
import jax.numpy as jnp
from jax import lax
from jax.experimental import pallas as pl
from jax.experimental.pallas import tpu as pltpu
from jax.experimental.pallas import tpu_sc as plsc

N = 10000
M = 320000
AF = 128
NF = 16

NC, NS = 2, 16
NWORK = NC * NS
GW = 80
SW1 = 2000

HALF = 5120
YW = 144
EB = 512
NEB = M // EB

_mesh = plsc.VectorSubcoreMesh(core_axis_name="c", subcore_axis_name="s")

_sc_params = pltpu.CompilerParams()
if "needs_layout_passes" in pltpu.CompilerParams.__dataclass_fields__:
    _sc_params = dataclasses.replace(_sc_params, needs_layout_passes=False)


def _k0_body(a_ref, wnode_ref, ps_ref, pn_ref):
    a = a_ref[...]
    ps_ref[...] = jnp.dot(a, wnode_ref[...][:, :256],
                          preferred_element_type=jnp.float32)
    pn_ref[...] = jnp.dot(a, wnode_ref[...][:, 256:],
                          preferred_element_type=jnp.float32)


def _make_tables(atom_in_fea, w_node):
    R = 400
    return pl.pallas_call(
        _k0_body,
        grid=(N // R,),
        in_specs=[
            pl.BlockSpec((R, AF), lambda i: (i, 0)),
            pl.BlockSpec((AF, 512), lambda i: (0, 0)),
        ],
        out_specs=[
            pl.BlockSpec((R, 256), lambda i: (i, 0)),
            pl.BlockSpec((R, 256), lambda i: (i, 0)),
        ],
        out_shape=[
            jax.ShapeDtypeStruct((N, 256), jnp.float32),
            jax.ShapeDtypeStruct((N, 256), jnp.float32),
        ],
    )(atom_in_fea, w_node)


@functools.partial(
    pl.kernel,
    mesh=_mesh,
    compiler_params=_sc_params,
    out_type=[
        jax.ShapeDtypeStruct((M, 256), jnp.float32),
        jax.ShapeDtypeStruct((M,), jnp.float32),
    ],
    scratch_types=[
        pltpu.VMEM((SW1,), jnp.int32),
        pltpu.VMEM((SW1,), jnp.int32),
        pltpu.VMEM((GW, 256), jnp.float32),
        pltpu.VMEM((GW, 256), jnp.float32),
        pltpu.VMEM((SW1,), jnp.float32),
        pltpu.VMEM((N,), jnp.float32),
        pltpu.SemaphoreType.DMA,
        pltpu.SemaphoreType.DMA,
    ],
)
def _k1_gather(ps_hbm, pn_hbm, si_hbm, ni_hbm, aw_hbm, xpre_hbm, wg_hbm,
               si_v, ni_v, a_v, b_v, w_v, wtab_v, sem_a, sem_b):
    wid = lax.axis_index("s") * NC + lax.axis_index("c")
    per_w = M // NWORK
    base = wid * per_w

    pltpu.sync_copy(aw_hbm, wtab_v)

    @pl.loop(0, per_w // SW1)
    def _sup(u):
        off0 = base + u * SW1
        pltpu.sync_copy(si_hbm.at[pl.ds(off0, SW1)], si_v)
        pltpu.sync_copy(ni_hbm.at[pl.ds(off0, SW1)], ni_v)

        @pl.loop(0, SW1, step=GW)
        def _win(sw):
            cp_a = pltpu.async_copy(ps_hbm.at[si_v.at[pl.ds(sw, GW)]], a_v,
                                    sem_a)
            cp_b = pltpu.async_copy(pn_hbm.at[ni_v.at[pl.ds(sw, GW)]], b_v,
                                    sem_b)
            cp_a.wait()
            cp_b.wait()

            @pl.loop(0, GW)
            def _row(r):
                @pl.loop(0, 256, step=16)
                def _chunk(cc):
                    b_v[r, pl.ds(cc, 16)] = (b_v[r, pl.ds(cc, 16)]
                                             + a_v[r, pl.ds(cc, 16)])

            @pl.loop(0, GW, step=16)
            def _wchunk(cc):
                idx = ni_v[pl.ds(sw + cc, 16)]
                w_v[pl.ds(sw + cc, 16)] = plsc.load_gather(wtab_v, [idx])

            pltpu.sync_copy(b_v, xpre_hbm.at[pl.ds(off0 + sw, GW)])

        pltpu.sync_copy(w_v, wg_hbm.at[pl.ds(off0, SW1)])


def _k2_body(xpre_ref, bond_ref, wb_ref, acc_ref):
    @pl.when(pl.program_id(0) == 0)
    def _init():
        acc_ref[...] = jnp.zeros_like(acc_ref)

    x = xpre_ref[...] + jnp.dot(
        bond_ref[...], wb_ref[...], preferred_element_type=jnp.float32)
    acc_ref[0:1, :] += jnp.sum(x, axis=0, keepdims=True)
    acc_ref[1:2, :] += jnp.sum(x * x, axis=0, keepdims=True)


def _bn_stats(xpre, bond, wb_cat):
    return pl.pallas_call(
        _k2_body,
        grid=(NEB,),
        in_specs=[
            pl.BlockSpec((EB, 256), lambda i: (i, 0)),
            pl.BlockSpec((EB, NF), lambda i: (i, 0)),
            pl.BlockSpec((NF, 256), lambda i: (0, 0)),
        ],
        out_specs=pl.BlockSpec((8, 256), lambda i: (0, 0)),
        out_shape=jax.ShapeDtypeStruct((8, 256), jnp.float32),
    )(xpre, bond, wb_cat)


def _k3_body(xpre_ref, bond_ref, wcol_ref, wb_ref, stats_ref, gb_ref, wg_ref,
             y_ref):
    inv_m = 1.0 / M
    mu = stats_ref[0:1, :] * inv_m
    ex2 = stats_ref[1:2, :] * inv_m
    var = ex2 - mu * mu
    rstd = lax.rsqrt(var + 1e-5)
    a_aff = rstd * gb_ref[0:1, :]
    c_aff = gb_ref[1:2, :] - mu * a_aff

    x = xpre_ref[...] + jnp.dot(
        bond_ref[...], wb_ref[...], preferred_element_type=jnp.float32)
    xn = x * a_aff + c_aff
    xf = xn[:, :AF]
    xc = xn[:, AF:]
    f = jax.nn.sigmoid(xf)
    e = jnp.where(xc > 0, xc, jnp.exp(jnp.minimum(xc, 0.0)) - 1.0)
    msg = f * e

    g = jnp.dot(msg, wg_ref[...], preferred_element_type=jnp.float32)
    g = g[:, 0:1] + gb_ref[2:3, 0:1]
    t = wcol_ref[...] * jnp.exp(jnp.minimum(g, 50.0))
    pad = jnp.zeros((msg.shape[0], YW - AF - 1), jnp.float32)
    y_ref[...] = jnp.concatenate([t * msg, t, pad], axis=1)


def _messages(xpre, bond, wcol, wb_cat, stats, gb, wg):
    return pl.pallas_call(
        _k3_body,
        grid=(NEB,),
        in_specs=[
            pl.BlockSpec((EB, 256), lambda i: (i, 0)),
            pl.BlockSpec((EB, NF), lambda i: (i, 0)),
            pl.BlockSpec((EB, 1), lambda i: (i, 0)),
            pl.BlockSpec((NF, 256), lambda i: (0, 0)),
            pl.BlockSpec((8, 256), lambda i: (0, 0)),
            pl.BlockSpec((8, 256), lambda i: (0, 0)),
            pl.BlockSpec((AF, 1), lambda i: (0, 0)),
        ],
        out_specs=pl.BlockSpec((EB, YW), lambda i: (i, 0)),
        out_shape=jax.ShapeDtypeStruct((M, YW), jnp.float32),
    )(xpre, bond, wcol, wb_cat, stats, gb, wg)


NPAD = 2 * HALF
NODES_PER_W = NPAD // NWORK
ACCTOT = NPAD
SW4 = 1600


@functools.partial(
    pl.kernel,
    mesh=_mesh,
    compiler_params=_sc_params,
    out_type=jax.ShapeDtypeStruct((NPAD, YW), jnp.float32),
    scratch_types=[
        pltpu.VMEM((GW, YW), jnp.float32),
        pltpu.VMEM((SW4,), jnp.int32),
        pltpu.VMEM((NODES_PER_W + 8, YW), jnp.float32),
    ],
)
def _k4_scatter(y_hbm, si_hbm, out_hbm, y_v, si_v, acc_v):
    cid = lax.axis_index("c")
    sid = lax.axis_index("s")
    wid = cid * NS + sid
    nlo = wid * NODES_PER_W
    nhi = nlo + NODES_PER_W
    ii16 = lax.iota(jnp.int32, 16)

    @pl.loop(0, NODES_PER_W + 8)
    def _zr(r):
        @pl.loop(0, YW, step=16)
        def _zc(cc):
            acc_v[r, pl.ds(cc, 16)] = jnp.zeros((16,), jnp.float32)

    @pl.loop(0, M // SW4)
    def _scan(s):
        soff = s * SW4
        pltpu.sync_copy(si_hbm.at[pl.ds(soff, SW4)], si_v)
        first = jnp.min(si_v[pl.ds(0, 16)])
        last = jnp.max(si_v[pl.ds(SW4 - 16, 16)])

        @pl.when(jnp.logical_and(last >= nlo, first < nhi))
        def _block():
            @pl.loop(0, SW4, step=GW)
            def _sub(sw):
                f2 = jnp.min(si_v[pl.ds(sw, 16)])
                l2 = jnp.max(si_v[pl.ds(sw + GW - 16, 16)])

                @pl.when(jnp.logical_and(l2 >= nlo, f2 < nhi))
                def _accum():
                    pltpu.sync_copy(y_hbm.at[pl.ds(soff + sw, GW)], y_v)

                    @pl.loop(0, GW, step=16)
                    def _ec(ec):
                        sic = si_v[pl.ds(sw + ec, 16)]
                        loc = sic - nlo
                        inr = jnp.logical_and(loc >= 0, loc < NODES_PER_W)
                        rowc = jnp.where(inr, loc, NODES_PER_W)
                        for e in range(16):
                            row_e = jnp.sum(jnp.where(ii16 == e, rowc, 0))
                            rows = jnp.broadcast_to(row_e, (16,))

                            @pl.loop(0, YW, step=16)
                            def _ck(k):
                                v = y_v[ec + e, pl.ds(k, 16)]
                                plsc.addupdate_scatter(
                                    acc_v, [rows, k + ii16], v)

    @pl.loop(0, NODES_PER_W // GW)
    def _dump(k):
        pltpu.sync_copy(acc_v.at[pl.ds(k * GW, GW)],
                        out_hbm.at[pl.ds(nlo + k * GW, GW)])


def _k5_body(parts_ref, out_ref):
    s = parts_ref[...]
    out_ref[...] = s[:, :AF] / (s[:, AF:AF + 1] + 1e-13)


def _finalize(parts):
    R = 80
    return pl.pallas_call(
        _k5_body,
        grid=(N // R,),
        in_specs=[pl.BlockSpec((R, YW), lambda i: (i, 0))],
        out_specs=pl.BlockSpec((R, AF), lambda i: (i, 0)),
        out_shape=jax.ShapeDtypeStruct((N, AF), jnp.float32),
    )(parts)


def kernel(atom_weights, atom_in_fea, bond_nbr_fea, self_fea_idx, nbr_fea_idx,
           W_filter, b_filter, gamma_filter, beta_filter,
           W_core, b_core, gamma_core, beta_core,
           W_gate, b_gate):
    f32 = jnp.float32
    w_node = jnp.concatenate(
        [W_filter[:AF], W_core[:AF], W_filter[AF:2 * AF], W_core[AF:2 * AF]],
        axis=1).astype(f32)
    wb_cat = jnp.concatenate(
        [W_filter[2 * AF:], W_core[2 * AF:]], axis=1).astype(f32)
    gamma_cat = jnp.concatenate([gamma_filter, gamma_core])[None, :]
    beta_cat = jnp.concatenate([beta_filter, beta_core])[None, :]
    bg_row = jnp.broadcast_to(b_gate.reshape(1, 1), (1, 256))
    gb = jnp.concatenate(
        [gamma_cat, beta_cat, bg_row,
         jnp.zeros((5, 256), f32)], axis=0).astype(f32)

    ps, pn = _make_tables(atom_in_fea.astype(f32), w_node)
    xpre, wg = _k1_gather(ps, pn, self_fea_idx, nbr_fea_idx,
                          atom_weights.reshape(N).astype(f32))
    bond = bond_nbr_fea.astype(f32)
    stats = _bn_stats(xpre, bond, wb_cat)
    y = _messages(xpre, bond, wg.reshape(M, 1), wb_cat, stats, gb,
                  W_gate.astype(f32))
    parts = _k4_scatter(y, self_fea_idx)
    return _finalize(parts)


# speedup vs baseline: 3.3968x; 1.1770x over previous
import dataclasses
import functools

import jax
import jax.numpy as jnp
from jax import lax
from jax.experimental import pallas as pl
from jax.experimental.pallas import tpu as pltpu
from jax.experimental.pallas import tpu_sc as plsc

N = 10000
M = 320000
AF = 128
NF = 16

NC, NS = 2, 16
NWORK = NC * NS
GW = 80
SW1 = 2000

HALF = 5120
YW = 144
EB = 512
NEB = M // EB

_mesh = plsc.VectorSubcoreMesh(core_axis_name="c", subcore_axis_name="s")

_sc_params = pltpu.CompilerParams()
if "needs_layout_passes" in pltpu.CompilerParams.__dataclass_fields__:
    _sc_params = dataclasses.replace(_sc_params, needs_layout_passes=False)


def _k0_body(a_ref, wnode_ref, ps_ref, pn_ref):
    a = a_ref[...]
    ps_ref[...] = jnp.dot(a, wnode_ref[...][:, :256],
                          preferred_element_type=jnp.float32)
    pn_ref[...] = jnp.dot(a, wnode_ref[...][:, 256:],
                          preferred_element_type=jnp.float32)


def _make_tables(atom_in_fea, w_node):
    R = 400
    return pl.pallas_call(
        _k0_body,
        grid=(N // R,),
        in_specs=[
            pl.BlockSpec((R, AF), lambda i: (i, 0)),
            pl.BlockSpec((AF, 512), lambda i: (0, 0)),
        ],
        out_specs=[
            pl.BlockSpec((R, 256), lambda i: (i, 0)),
            pl.BlockSpec((R, 256), lambda i: (i, 0)),
        ],
        out_shape=[
            jax.ShapeDtypeStruct((N, 256), jnp.float32),
            jax.ShapeDtypeStruct((N, 256), jnp.float32),
        ],
    )(atom_in_fea, w_node)


@functools.partial(
    pl.kernel,
    mesh=_mesh,
    compiler_params=_sc_params,
    out_type=[
        jax.ShapeDtypeStruct((M, 256), jnp.float32),
        jax.ShapeDtypeStruct((M,), jnp.float32),
    ],
    scratch_types=[
        pltpu.VMEM((SW1,), jnp.int32),
        pltpu.VMEM((SW1,), jnp.int32),
        pltpu.VMEM((GW, 256), jnp.float32),
        pltpu.VMEM((GW, 256), jnp.float32),
        pltpu.VMEM((GW, 256), jnp.float32),
        pltpu.VMEM((GW, 256), jnp.float32),
        pltpu.VMEM((SW1,), jnp.float32),
        pltpu.VMEM((N,), jnp.float32),
        pltpu.SemaphoreType.DMA,
        pltpu.SemaphoreType.DMA,
        pltpu.SemaphoreType.DMA,
        pltpu.SemaphoreType.DMA,
    ],
)
def _k1_gather(ps_hbm, pn_hbm, si_hbm, ni_hbm, aw_hbm, xpre_hbm, wg_hbm,
               si_v, ni_v, a0_v, b0_v, a1_v, b1_v, w_v, wtab_v,
               sa0, sb0, sa1, sb1):
    wid = lax.axis_index("s") * NC + lax.axis_index("c")
    per_w = M // NWORK
    base = wid * per_w
    nwin = SW1 // GW

    pltpu.sync_copy(aw_hbm, wtab_v)

    @pl.loop(0, per_w // SW1)
    def _sup(u):
        off0 = base + u * SW1
        pltpu.sync_copy(si_hbm.at[pl.ds(off0, SW1)], si_v)
        pltpu.sync_copy(ni_hbm.at[pl.ds(off0, SW1)], ni_v)

        def _issue(sw, a_v, b_v, sem_a, sem_b):
            pltpu.async_copy(ps_hbm.at[si_v.at[pl.ds(sw, GW)]], a_v, sem_a)
            pltpu.async_copy(pn_hbm.at[ni_v.at[pl.ds(sw, GW)]], b_v, sem_b)

        def _wait(a_v, b_v, sem_a, sem_b):
            pltpu.make_async_copy(ps_hbm.at[si_v.at[pl.ds(0, GW)]], a_v,
                                  sem_a).wait()
            pltpu.make_async_copy(pn_hbm.at[ni_v.at[pl.ds(0, GW)]], b_v,
                                  sem_b).wait()

        def _consume(sw, a_v, b_v):
            @pl.loop(0, GW)
            def _row(r):
                @pl.loop(0, 256, step=16)
                def _chunk(cc):
                    b_v[r, pl.ds(cc, 16)] = (b_v[r, pl.ds(cc, 16)]
                                             + a_v[r, pl.ds(cc, 16)])

            @pl.loop(0, GW, step=16)
            def _wchunk(cc):
                idx = ni_v[pl.ds(sw + cc, 16)]
                w_v[pl.ds(sw + cc, 16)] = plsc.load_gather(wtab_v, [idx])

            pltpu.sync_copy(b_v, xpre_hbm.at[pl.ds(off0 + sw, GW)])

        _issue(0, a0_v, b0_v, sa0, sb0)

        @pl.loop(0, (nwin - 1) // 2)
        def _dw(i):
            sw0 = i * (2 * GW)
            _issue(sw0 + GW, a1_v, b1_v, sa1, sb1)
            _wait(a0_v, b0_v, sa0, sb0)
            _consume(sw0, a0_v, b0_v)
            _issue(sw0 + 2 * GW, a0_v, b0_v, sa0, sb0)
            _wait(a1_v, b1_v, sa1, sb1)
            _consume(sw0 + GW, a1_v, b1_v)

        _wait(a0_v, b0_v, sa0, sb0)
        _consume(SW1 - GW, a0_v, b0_v)

        pltpu.sync_copy(w_v, wg_hbm.at[pl.ds(off0, SW1)])


def _k2_body(xpre_ref, bond_ref, wb_ref, acc_ref):
    @pl.when(pl.program_id(0) == 0)
    def _init():
        acc_ref[...] = jnp.zeros_like(acc_ref)

    x = xpre_ref[...] + jnp.dot(
        bond_ref[...], wb_ref[...], preferred_element_type=jnp.float32)
    acc_ref[0:1, :] += jnp.sum(x, axis=0, keepdims=True)
    acc_ref[1:2, :] += jnp.sum(x * x, axis=0, keepdims=True)


def _bn_stats(xpre, bond, wb_cat):
    return pl.pallas_call(
        _k2_body,
        grid=(NEB,),
        in_specs=[
            pl.BlockSpec((EB, 256), lambda i: (i, 0)),
            pl.BlockSpec((EB, NF), lambda i: (i, 0)),
            pl.BlockSpec((NF, 256), lambda i: (0, 0)),
        ],
        out_specs=pl.BlockSpec((8, 256), lambda i: (0, 0)),
        out_shape=jax.ShapeDtypeStruct((8, 256), jnp.float32),
    )(xpre, bond, wb_cat)


def _k3_body(xpre_ref, bond_ref, wcol_ref, wb_ref, stats_ref, gb_ref, wg_ref,
             y_ref):
    inv_m = 1.0 / M
    mu = stats_ref[0:1, :] * inv_m
    ex2 = stats_ref[1:2, :] * inv_m
    var = ex2 - mu * mu
    rstd = lax.rsqrt(var + 1e-5)
    a_aff = rstd * gb_ref[0:1, :]
    c_aff = gb_ref[1:2, :] - mu * a_aff

    x = xpre_ref[...] + jnp.dot(
        bond_ref[...], wb_ref[...], preferred_element_type=jnp.float32)
    xn = x * a_aff + c_aff
    xf = xn[:, :AF]
    xc = xn[:, AF:]
    f = jax.nn.sigmoid(xf)
    e = jnp.where(xc > 0, xc, jnp.exp(jnp.minimum(xc, 0.0)) - 1.0)
    msg = f * e

    g = jnp.dot(msg, wg_ref[...], preferred_element_type=jnp.float32)
    g = g[:, 0:1] + gb_ref[2:3, 0:1]
    t = wcol_ref[...] * jnp.exp(jnp.minimum(g, 50.0))
    pad = jnp.zeros((msg.shape[0], YW - AF - 1), jnp.float32)
    y_ref[...] = jnp.concatenate([t * msg, t, pad], axis=1)


def _messages(xpre, bond, wcol, wb_cat, stats, gb, wg):
    return pl.pallas_call(
        _k3_body,
        grid=(NEB,),
        in_specs=[
            pl.BlockSpec((EB, 256), lambda i: (i, 0)),
            pl.BlockSpec((EB, NF), lambda i: (i, 0)),
            pl.BlockSpec((EB, 1), lambda i: (i, 0)),
            pl.BlockSpec((NF, 256), lambda i: (0, 0)),
            pl.BlockSpec((8, 256), lambda i: (0, 0)),
            pl.BlockSpec((8, 256), lambda i: (0, 0)),
            pl.BlockSpec((AF, 1), lambda i: (0, 0)),
        ],
        out_specs=pl.BlockSpec((EB, YW), lambda i: (i, 0)),
        out_shape=jax.ShapeDtypeStruct((M, YW), jnp.float32),
    )(xpre, bond, wcol, wb_cat, stats, gb, wg)


NPAD = 2 * HALF
NODES_PER_W = NPAD // NWORK
ACCTOT = NPAD
SW4 = 1600


@functools.partial(
    pl.kernel,
    mesh=_mesh,
    compiler_params=_sc_params,
    out_type=jax.ShapeDtypeStruct((NPAD, YW), jnp.float32),
    scratch_types=[
        pltpu.VMEM((GW, YW), jnp.float32),
        pltpu.VMEM((SW4,), jnp.int32),
        pltpu.VMEM((NODES_PER_W + 8, YW), jnp.float32),
    ],
)
def _k4_scatter(y_hbm, si_hbm, out_hbm, y_v, si_v, acc_v):
    cid = lax.axis_index("c")
    sid = lax.axis_index("s")
    wid = cid * NS + sid
    nlo = wid * NODES_PER_W
    nhi = nlo + NODES_PER_W
    ii16 = lax.iota(jnp.int32, 16)

    @pl.loop(0, NODES_PER_W + 8)
    def _zr(r):
        @pl.loop(0, YW, step=16)
        def _zc(cc):
            acc_v[r, pl.ds(cc, 16)] = jnp.zeros((16,), jnp.float32)

    @pl.loop(0, M // SW4)
    def _scan(s):
        soff = s * SW4
        pltpu.sync_copy(si_hbm.at[pl.ds(soff, SW4)], si_v)
        first = jnp.min(si_v[pl.ds(0, 16)])
        last = jnp.max(si_v[pl.ds(SW4 - 16, 16)])

        @pl.when(jnp.logical_and(last >= nlo, first < nhi))
        def _block():
            @pl.loop(0, SW4, step=GW)
            def _sub(sw):
                f2 = jnp.min(si_v[pl.ds(sw, 16)])
                l2 = jnp.max(si_v[pl.ds(sw + GW - 16, 16)])

                @pl.when(jnp.logical_and(l2 >= nlo, f2 < nhi))
                def _accum():
                    pltpu.sync_copy(y_hbm.at[pl.ds(soff + sw, GW)], y_v)

                    @pl.loop(0, GW, step=16)
                    def _ec(ec):
                        sic = si_v[pl.ds(sw + ec, 16)]
                        loc = sic - nlo
                        inr = jnp.logical_and(loc >= 0, loc < NODES_PER_W)
                        rowc = jnp.where(inr, loc, NODES_PER_W)
                        for e in range(16):
                            row_e = jnp.sum(jnp.where(ii16 == e, rowc, 0))
                            rows = jnp.broadcast_to(row_e, (16,))

                            @pl.loop(0, YW, step=16)
                            def _ck(k):
                                v = y_v[ec + e, pl.ds(k, 16)]
                                plsc.addupdate_scatter(
                                    acc_v, [rows, k + ii16], v)

    @pl.loop(0, NODES_PER_W // GW)
    def _dump(k):
        pltpu.sync_copy(acc_v.at[pl.ds(k * GW, GW)],
                        out_hbm.at[pl.ds(nlo + k * GW, GW)])


def _k5_body(parts_ref, out_ref):
    s = parts_ref[...]
    out_ref[...] = s[:, :AF] / (s[:, AF:AF + 1] + 1e-13)


def _finalize(parts):
    R = 80
    return pl.pallas_call(
        _k5_body,
        grid=(N // R,),
        in_specs=[pl.BlockSpec((R, YW), lambda i: (i, 0))],
        out_specs=pl.BlockSpec((R, AF), lambda i: (i, 0)),
        out_shape=jax.ShapeDtypeStruct((N, AF), jnp.float32),
    )(parts)


def kernel(atom_weights, atom_in_fea, bond_nbr_fea, self_fea_idx, nbr_fea_idx,
           W_filter, b_filter, gamma_filter, beta_filter,
           W_core, b_core, gamma_core, beta_core,
           W_gate, b_gate):
    f32 = jnp.float32
    w_node = jnp.concatenate(
        [W_filter[:AF], W_core[:AF], W_filter[AF:2 * AF], W_core[AF:2 * AF]],
        axis=1).astype(f32)
    wb_cat = jnp.concatenate(
        [W_filter[2 * AF:], W_core[2 * AF:]], axis=1).astype(f32)
    gamma_cat = jnp.concatenate([gamma_filter, gamma_core])[None, :]
    beta_cat = jnp.concatenate([beta_filter, beta_core])[None, :]
    bg_row = jnp.broadcast_to(b_gate.reshape(1, 1), (1, 256))
    gb = jnp.concatenate(
        [gamma_cat, beta_cat, bg_row,
         jnp.zeros((5, 256), f32)], axis=0).astype(f32)

    ps, pn = _make_tables(atom_in_fea.astype(f32), w_node)
    xpre, wg = _k1_gather(ps, pn, self_fea_idx, nbr_fea_idx,
                          atom_weights.reshape(N).astype(f32))
    bond = bond_nbr_fea.astype(f32)
    stats = _bn_stats(xpre, bond, wb_cat)
    y = _messages(xpre, bond, wg.reshape(M, 1), wb_cat, stats, gb,
                  W_gate.astype(f32))
    parts = _k4_scatter(y, self_fea_idx)
    return _finalize(parts)
